# Initial kernel scaffold; baseline (speedup 1.0000x reference)
import jax
import jax.numpy as jnp
from jax.experimental import pallas as pl


def kernel(inputs, embeddings):
    raise NotImplementedError("write your pallas kernel here")


# speedup vs baseline: 4.5422x; 4.5422x over previous
import functools

import jax
import jax.numpy as jnp
from jax import lax
from jax.experimental import pallas as pl
from jax.experimental.pallas import tpu as pltpu
from jax.experimental.pallas import tpu_sc as plsc

_VOCAB = 100000
_EMB_DIM = 64
_DEFAULT_IDX = 0
_LANES = 16
_SEG = 128
_GROUP = 5
_NC = 2
_NS = 16


def _emb_gather(table, idx):
    n_total = idx.shape[0]
    nw = _NC * _NS
    per_w = n_total // nw
    n_groups = per_w // (_SEG * _GROUP)
    mesh = plsc.VectorSubcoreMesh(core_axis_name="c", subcore_axis_name="s")

    @functools.partial(
        pl.kernel,
        out_type=jax.ShapeDtypeStruct((n_total, _EMB_DIM), jnp.float32),
        mesh=mesh,
        scratch_types=[
            pltpu.VMEM((per_w,), jnp.int32),
            pltpu.VMEM((_SEG * _GROUP, _EMB_DIM), jnp.float32),
            pltpu.SemaphoreType.DMA,
        ],
        compiler_params=pltpu.CompilerParams(use_tc_tiling_on_sc=False),
    )
    def k(table_hbm, idx_hbm, out_hbm, idx_v, rows_v, sem):
        wid = lax.axis_index("c") * _NS + lax.axis_index("s")
        base = wid * per_w
        pltpu.sync_copy(idx_hbm.at[pl.ds(base, per_w)], idx_v)

        def clamp_body(i, carry):
            sl = pl.ds(i * _LANES, _LANES)
            v = idx_v[sl]
            ok = (v >= 0) & (v < _VOCAB)
            idx_v[sl] = jnp.where(ok, v, _DEFAULT_IDX)
            return carry

        lax.fori_loop(0, per_w // _LANES, clamp_body, 0)

        def group_body(g, carry):
            g0 = g * _SEG * _GROUP
            copies = [
                pltpu.async_copy(
                    table_hbm.at[idx_v.at[pl.ds(g0 + j * _SEG, _SEG)]],
                    rows_v.at[pl.ds(j * _SEG, _SEG)],
                    sem)
                for j in range(_GROUP)
            ]
            for c in copies:
                c.wait()
            pltpu.sync_copy(
                rows_v, out_hbm.at[pl.ds(base + g0, _SEG * _GROUP)])
            return carry

        lax.fori_loop(0, n_groups, group_body, 0)

    return k(table, idx)


def kernel(inputs, embeddings):
    b, h = inputs.shape
    out = _emb_gather(embeddings, inputs.reshape(-1))
    return out.reshape(b, h, _EMB_DIM)
